# baseline (speedup 1.0000x reference)
import jax
import jax.numpy as jnp
from jax import lax
from jax.experimental import pallas as pl
from jax.experimental.pallas import tpu as pltpu
from jax.experimental.pallas import tpu_sc as plsc

_VOCAB = 100000
_EMBED = 32
_NUM_TOKENS = 26
_B = 4096
_L = 50

_IPC = _NUM_TOKENS // 2
_CW = 512
_NBC = _B // _CW
_NCH = 6 * _NBC
_LB = 8 * _CW * 4
_SB = 8 * _CW * 4
_TB = 2 * _CW * 4


def _mt_body(tt, xt, out, trow, xch0, xch1, stg0, stg1, xsh,
             xlsem0, xlsem1, ssem0, ssem1):
    sc = lax.axis_index("c")
    tec = lax.axis_index("s")

    @pl.when(tec == 0)
    def _zero_pad():
        zv = jnp.zeros((16,), jnp.int32)
        for l8 in range(8):
            for v in range(_CW // 16):
                xch0[l8, pl.ds(v * 16, 16)] = zv
        for bc in range(_NBC):
            pltpu.sync_copy(xch0, xsh.at[pl.ds(48, 8), pl.ds(bc * _CW, _CW)])

    def compute_chunk(xch, stg):
        for l8 in range(8):
            for g in range(0, _CW // 16, 32):
                idxs = [xch[l8, pl.ds((g + j) * 16, 16)] for j in range(32)]
                vals = [plsc.load_gather(trow, [ix]) for ix in idxs]
                for j in range(32):
                    stg[l8, 0, pl.ds((g + j) * 16, 16)] = vals[j]

    def ii_body(ii, carry):
        i = sc * _IPC + ii

        @pl.when(tec == 0)
        def _stage_x():
            pltpu.sync_copy(xt.at[i, pl.ds(0, 48), :], xsh.at[pl.ds(0, 48), :])
            pltpu.sync_copy(xt.at[i, pl.ds(48, 2), :], xsh.at[pl.ds(48, 2), :])

        plsc.subcore_barrier()

        for rep in range(2):
            e = 16 * rep + tec
            c = i * _EMBED + e
            pltpu.sync_copy(tt.at[i, e, :], trow)

            def load_chunk(ch, xch, sem):
                lt = ch // _NBC
                bc = ch % _NBC
                pltpu.async_copy(
                    xsh.at[pl.ds(lt * 8, 8), pl.ds(bc * _CW, _CW)], xch, sem)

            def store_chunk(ch, stg, sem):
                lt = ch // _NBC
                bc = ch % _NBC
                pltpu.async_copy(
                    stg,
                    out.at[pl.ds(lt * 8, 8), pl.ds(c, 1), pl.ds(bc * _CW, _CW)],
                    sem)

            def drain(sem, nbytes):
                pltpu.make_async_copy(
                    out.at[pl.ds(0, nbytes // (_CW * 4)), pl.ds(0, 1),
                           pl.ds(0, _CW)],
                    stg0.at[pl.ds(0, nbytes // (_CW * 4))], sem).wait()

            load_chunk(0, xch0, xlsem0)
            load_chunk(1, xch1, xlsem1)

            def pair(m, carry2):
                ch0 = 2 * m
                pltpu.make_async_copy(xsh.at[pl.ds(0, 8), pl.ds(0, _CW)],
                                      xch0, xlsem0).wait()

                @pl.when(m > 0)
                def _():
                    drain(ssem0, _SB)

                compute_chunk(xch0, stg0)
                load_chunk(ch0 + 2, xch0, xlsem0)
                store_chunk(ch0, stg0, ssem0)
                pltpu.make_async_copy(xsh.at[pl.ds(0, 8), pl.ds(0, _CW)],
                                      xch1, xlsem1).wait()

                @pl.when(m > 0)
                def _():
                    drain(ssem1, _SB)

                compute_chunk(xch1, stg1)
                load_chunk(ch0 + 3, xch1, xlsem1)
                store_chunk(ch0 + 1, stg1, ssem1)
                return carry2

            lax.fori_loop(0, _NCH // 2, pair, 0)

            pltpu.make_async_copy(xsh.at[pl.ds(0, 8), pl.ds(0, _CW)],
                                  xch0, xlsem0).wait()
            pltpu.make_async_copy(xsh.at[pl.ds(0, 8), pl.ds(0, _CW)],
                                  xch1, xlsem1).wait()
            drain(ssem0, _SB)
            drain(ssem1, _SB)

            for bc in range(_NBC):
                if bc > 0:
                    drain(ssem0, _TB)
                pltpu.sync_copy(xsh.at[pl.ds(48, 2), pl.ds(bc * _CW, _CW)],
                                xch0.at[pl.ds(0, 2)])
                for l8 in range(2):
                    for g in range(0, _CW // 16, 8):
                        idxs = [xch0[l8, pl.ds((g + j) * 16, 16)]
                                for j in range(8)]
                        vals = [plsc.load_gather(trow, [ix]) for ix in idxs]
                        for j in range(8):
                            stg0[l8, 0, pl.ds((g + j) * 16, 16)] = vals[j]
                pltpu.async_copy(
                    stg0.at[pl.ds(0, 2)],
                    out.at[pl.ds(48, 2), pl.ds(c, 1), pl.ds(bc * _CW, _CW)],
                    ssem0)
            drain(ssem0, _TB)

        plsc.subcore_barrier()
        return carry

    lax.fori_loop(0, _IPC, ii_body, 0)


@jax.jit
def _mt_gather(tt, xt):
    mesh = plsc.VectorSubcoreMesh(core_axis_name="c", subcore_axis_name="s")
    return pl.kernel(
        _mt_body,
        out_type=jax.ShapeDtypeStruct((_L, _NUM_TOKENS * _EMBED, _B),
                                      jnp.float32),
        mesh=mesh,
        compiler_params=pltpu.CompilerParams(
            use_tc_tiling_on_sc=True, needs_layout_passes=False),
        scratch_types=[
            pltpu.VMEM((_VOCAB,), jnp.float32),
            pltpu.VMEM((8, _CW), jnp.int32),
            pltpu.VMEM((8, _CW), jnp.int32),
            pltpu.VMEM((8, 1, _CW), jnp.float32),
            pltpu.VMEM((8, 1, _CW), jnp.float32),
            pltpu.VMEM_SHARED((56, _B), jnp.int32),
            pltpu.SemaphoreType.DMA,
            pltpu.SemaphoreType.DMA,
            pltpu.SemaphoreType.DMA,
            pltpu.SemaphoreType.DMA,
        ],
    )(tt, xt)


def kernel(x, tables):
    xt = jnp.transpose(x, (2, 1, 0))
    tt = jnp.transpose(tables, (0, 2, 1))
    out_t = _mt_gather(tt, xt)
    return jnp.transpose(out_t, (2, 0, 1))


# speedup vs baseline: 1.0241x; 1.0241x over previous
import jax
import jax.numpy as jnp
from jax import lax
from jax.experimental import pallas as pl
from jax.experimental.pallas import tpu as pltpu
from jax.experimental.pallas import tpu_sc as plsc

_VOCAB = 100000
_EMBED = 32
_NUM_TOKENS = 26
_B = 4096
_L = 50

_IPC = _NUM_TOKENS // 2
_CW = 512
_NBC = _B // _CW
_NCH = 6 * _NBC
_LB = 8 * _CW * 4
_SB = 8 * _CW * 4
_TB = 2 * _CW * 4


def _mt_body(tt, xt, out, trow, xch0, xch1, stg0, stg1, xsh,
             xlsem0, xlsem1, ssem0, ssem1):
    sc = lax.axis_index("c")
    tec = lax.axis_index("s")

    @pl.when(tec == 0)
    def _zero_pad():
        zv = jnp.zeros((16,), jnp.int32)
        for l8 in range(8):
            for v in range(_CW // 16):
                xch0[l8, pl.ds(v * 16, 16)] = zv
        for bc in range(_NBC):
            pltpu.sync_copy(xch0, xsh.at[pl.ds(48, 8), pl.ds(bc * _CW, _CW)])

    def compute_chunk(xch, stg):
        nv = _CW // 16
        for g in range(0, 8 * nv, 16):
            pos = [((g + j) // nv, (g + j) % nv) for j in range(16)]
            idxs = [xch[l8, pl.ds(bv * 16, 16)] for l8, bv in pos]
            vals = [plsc.load_gather(trow, [ix]) for ix in idxs]
            for j, (l8, bv) in enumerate(pos):
                stg[l8, 0, pl.ds(bv * 16, 16)] = vals[j]

    def ii_body(ii, carry):
        i = sc * _IPC + ii

        @pl.when(tec == 0)
        def _stage_x():
            pltpu.sync_copy(xt.at[i, pl.ds(0, 48), :], xsh.at[pl.ds(0, 48), :])
            pltpu.sync_copy(xt.at[i, pl.ds(48, 2), :], xsh.at[pl.ds(48, 2), :])

        plsc.subcore_barrier()

        for rep in range(2):
            e = 16 * rep + tec
            c = i * _EMBED + e
            pltpu.sync_copy(tt.at[i, e, :], trow)

            def load_chunk(ch, xch, sem):
                lt = ch // _NBC
                bc = ch % _NBC
                pltpu.async_copy(
                    xsh.at[pl.ds(lt * 8, 8), pl.ds(bc * _CW, _CW)], xch, sem)

            def store_chunk(ch, stg, sem):
                lt = ch // _NBC
                bc = ch % _NBC
                pltpu.async_copy(
                    stg,
                    out.at[pl.ds(lt * 8, 8), pl.ds(c, 1), pl.ds(bc * _CW, _CW)],
                    sem)

            def drain(sem, nbytes):
                pltpu.make_async_copy(
                    out.at[pl.ds(0, nbytes // (_CW * 4)), pl.ds(0, 1),
                           pl.ds(0, _CW)],
                    stg0.at[pl.ds(0, nbytes // (_CW * 4))], sem).wait()

            load_chunk(0, xch0, xlsem0)
            load_chunk(1, xch1, xlsem1)

            def pair(m, carry2):
                ch0 = 2 * m
                pltpu.make_async_copy(xsh.at[pl.ds(0, 8), pl.ds(0, _CW)],
                                      xch0, xlsem0).wait()

                @pl.when(m > 0)
                def _():
                    drain(ssem0, _SB)

                compute_chunk(xch0, stg0)
                load_chunk(ch0 + 2, xch0, xlsem0)
                store_chunk(ch0, stg0, ssem0)
                pltpu.make_async_copy(xsh.at[pl.ds(0, 8), pl.ds(0, _CW)],
                                      xch1, xlsem1).wait()

                @pl.when(m > 0)
                def _():
                    drain(ssem1, _SB)

                compute_chunk(xch1, stg1)
                load_chunk(ch0 + 3, xch1, xlsem1)
                store_chunk(ch0 + 1, stg1, ssem1)
                return carry2

            lax.fori_loop(0, _NCH // 2, pair, 0)

            pltpu.make_async_copy(xsh.at[pl.ds(0, 8), pl.ds(0, _CW)],
                                  xch0, xlsem0).wait()
            pltpu.make_async_copy(xsh.at[pl.ds(0, 8), pl.ds(0, _CW)],
                                  xch1, xlsem1).wait()
            drain(ssem0, _SB)
            drain(ssem1, _SB)

            for bc in range(_NBC):
                if bc > 0:
                    drain(ssem0, _TB)
                pltpu.sync_copy(xsh.at[pl.ds(48, 2), pl.ds(bc * _CW, _CW)],
                                xch0.at[pl.ds(0, 2)])
                for l8 in range(2):
                    for g in range(0, _CW // 16, 8):
                        idxs = [xch0[l8, pl.ds((g + j) * 16, 16)]
                                for j in range(8)]
                        vals = [plsc.load_gather(trow, [ix]) for ix in idxs]
                        for j in range(8):
                            stg0[l8, 0, pl.ds((g + j) * 16, 16)] = vals[j]
                pltpu.async_copy(
                    stg0.at[pl.ds(0, 2)],
                    out.at[pl.ds(48, 2), pl.ds(c, 1), pl.ds(bc * _CW, _CW)],
                    ssem0)
            drain(ssem0, _TB)

        plsc.subcore_barrier()
        return carry

    lax.fori_loop(0, _IPC, ii_body, 0)


@jax.jit
def _mt_gather(tt, xt):
    mesh = plsc.VectorSubcoreMesh(core_axis_name="c", subcore_axis_name="s")
    return pl.kernel(
        _mt_body,
        out_type=jax.ShapeDtypeStruct((_L, _NUM_TOKENS * _EMBED, _B),
                                      jnp.float32),
        mesh=mesh,
        compiler_params=pltpu.CompilerParams(
            use_tc_tiling_on_sc=True, needs_layout_passes=False),
        scratch_types=[
            pltpu.VMEM((_VOCAB,), jnp.float32),
            pltpu.VMEM((8, _CW), jnp.int32),
            pltpu.VMEM((8, _CW), jnp.int32),
            pltpu.VMEM((8, 1, _CW), jnp.float32),
            pltpu.VMEM((8, 1, _CW), jnp.float32),
            pltpu.VMEM_SHARED((56, _B), jnp.int32),
            pltpu.SemaphoreType.DMA,
            pltpu.SemaphoreType.DMA,
            pltpu.SemaphoreType.DMA,
            pltpu.SemaphoreType.DMA,
        ],
    )(tt, xt)


def kernel(x, tables):
    xt = jnp.transpose(x, (2, 1, 0))
    tt = jnp.transpose(tables, (0, 2, 1))
    out_t = _mt_gather(tt, xt)
    return jnp.transpose(out_t, (2, 0, 1))
